# Initial kernel scaffold; baseline (speedup 1.0000x reference)
import jax
import jax.numpy as jnp
from jax.experimental import pallas as pl


def kernel(X, wave_codebook, mfcc_codebook):
    raise NotImplementedError("write your pallas kernel here")


# speedup vs baseline: 1.6660x; 1.6660x over previous
import functools

import jax
import jax.numpy as jnp
from jax import lax
from jax.experimental import pallas as pl
from jax.experimental.pallas import tpu as pltpu
from jax.experimental.pallas import tpu_sc as plsc

_NUM_EMB = 8192
_EMB_DIM = 128
_MFCC_EMB = _NUM_EMB // 4
_MFCC_DIM = 38
_COMMIT = 0.25

_R = 256

_INTERPRET = False


_CHUNKS_W = ((0, 2048), (2048, 2048), (4096, 2048), (6144, 2048))
_CHUNKS_M = ((0, 1024), (1024, 1024))


def _bf16_rne(x):
    u = lax.bitcast_convert_type(x, jnp.uint32)
    lsb = (u >> jnp.uint32(16)) & jnp.uint32(1)
    u = (u + jnp.uint32(0x7FFF) + lsb) & jnp.uint32(0xFFFF0000)
    return lax.bitcast_convert_type(u, jnp.float32)


def _chunk_argmin(blk, base):
    big = jnp.int32(2 ** 30)
    r, w = blk.shape
    acc_v = blk[:, 0:128]
    acc_k = jnp.zeros((r, 128), jnp.int32)
    for k in range(1, w // 128):
        v = blk[:, 128 * k:128 * (k + 1)]
        m = v < acc_v
        acc_v = jnp.where(m, v, acc_v)
        acc_k = jnp.where(m, jnp.int32(k), acc_k)
    lane = lax.broadcasted_iota(jnp.int32, (r, 128), 1)
    idx_full = acc_k * 128 + lane + base
    cv = jnp.min(acc_v, axis=1)
    ci = jnp.min(jnp.where(acc_v == cv[:, None], idx_full, big), axis=1)
    return cv, ci


def _chunked_argmin(s_mat, x2, chunks):
    acc_i = qv = raw = None
    for (s, w) in chunks:
        cv, ci = _chunk_argmin(s_mat[:, s:s + w], s)
        cv = x2 + cv
        if acc_i is None:
            acc_i, qv, raw = ci, cv, cv
        else:
            win = cv < qv
            acc_i = jnp.where(win, ci, acc_i)
            qv = jnp.where(win, cv, qv)
            raw = jnp.minimum(raw, cv)
        qv = _bf16_rne(qv)
    return acc_i, raw


def _vq_body(flat_ref, wcb_ref, mcb_ref, c_ref,
             widx_ref, wmin_ref, midx_ref, mmin_ref,
             e2w_ref, e2m_ref):
    i = pl.program_id(0)

    @pl.when(i == 0)
    def _():
        wcb = wcb_ref[...]
        e2w_ref[...] = 0.25 * jnp.sum(wcb * wcb, axis=0)[None, :]
        mcb = mcb_ref[...]
        e2m_ref[...] = 0.25 * jnp.sum(mcb * mcb, axis=0)[None, :]

    x = flat_ref[...]
    x2 = jnp.sum(x * x, axis=1)
    xe2 = jnp.dot(x, wcb_ref[...],
                  preferred_element_type=jnp.float32)
    sw = e2w_ref[...] + xe2
    widx, wmin = _chunked_argmin(sw, x2, _CHUNKS_W)
    widx_ref[...] = widx
    wmin_ref[...] = wmin

    m = jnp.dot(x, c_ref[...], precision=lax.Precision.HIGHEST,
                preferred_element_type=jnp.float32)
    m2 = jnp.sum(m * m, axis=1)
    me2 = jnp.dot(m, mcb_ref[...],
                  preferred_element_type=jnp.float32)
    sm = e2m_ref[...] + me2
    midx, mmin = _chunked_argmin(sm, m2, _CHUNKS_M)
    midx_ref[...] = midx + _NUM_EMB
    mmin_ref[...] = mmin


def _vq_stage(flat, wcb, mcb, c):
    n = flat.shape[0]
    grid = (n // _R,)
    out_shapes = [
        jax.ShapeDtypeStruct((n,), jnp.int32),
        jax.ShapeDtypeStruct((n,), jnp.float32),
        jax.ShapeDtypeStruct((n,), jnp.int32),
        jax.ShapeDtypeStruct((n,), jnp.float32),
    ]
    return pl.pallas_call(
        _vq_body,
        grid=grid,
        in_specs=[
            pl.BlockSpec((_R, _EMB_DIM), lambda i: (i, 0)),
            pl.BlockSpec((_EMB_DIM, _NUM_EMB), lambda i: (0, 0)),
            pl.BlockSpec((_MFCC_DIM, _MFCC_EMB), lambda i: (0, 0)),
            pl.BlockSpec((_EMB_DIM, _MFCC_DIM), lambda i: (0, 0)),
        ],
        out_specs=[
            pl.BlockSpec((_R,), lambda i: (i,)),
            pl.BlockSpec((_R,), lambda i: (i,)),
            pl.BlockSpec((_R,), lambda i: (i,)),
            pl.BlockSpec((_R,), lambda i: (i,)),
        ],
        out_shape=out_shapes,
        scratch_shapes=[
            pltpu.VMEM((1, _NUM_EMB), jnp.float32),
            pltpu.VMEM((1, _MFCC_EMB), jnp.float32),
        ],
        compiler_params=pltpu.CompilerParams(
            dimension_semantics=("arbitrary",),
        ),
        interpret=_INTERPRET,
    )(flat, wcb, mcb, c)


_NW = 32
_SLAB = _NUM_EMB // _NW
_L = 16


def _hist_chunk(idx16, cnt_ref):
    iota = lax.iota(jnp.int32, _L)
    occ = None
    prior = None
    for j in range(1, _L):
        perm = lax.rem(iota + j, _L)
        rot = idx16.at[perm].get(mode="promise_in_bounds")
        eq = rot == idx16
        inc = jnp.where(eq, 1.0, 0.0)
        occ = inc if occ is None else occ + inc
        eq_prior = jnp.logical_and(eq, iota >= (_L - j))
        pinc = jnp.where(eq_prior, 1, 0)
        prior = pinc if prior is None else prior + pinc
    first = prior == 0
    plsc.addupdate_scatter(cnt_ref, [idx16], occ + 1.0, mask=first)


def _hist_sc(widx, midx, zeros):
    mesh = plsc.VectorSubcoreMesh(core_axis_name="c", subcore_axis_name="s")

    @functools.partial(
        pl.kernel,
        mesh=mesh,
        out_type=[
            jax.ShapeDtypeStruct((_NW, _NUM_EMB), jnp.float32),
            jax.ShapeDtypeStruct((_NW, _MFCC_EMB), jnp.float32),
        ],
        scratch_types=[
            pltpu.VMEM((_SLAB,), jnp.int32),
            pltpu.VMEM((_SLAB,), jnp.int32),
            pltpu.VMEM((_NUM_EMB,), jnp.float32),
            pltpu.VMEM((_MFCC_EMB,), jnp.float32),
        ],
        compiler_params=pltpu.CompilerParams(needs_layout_passes=False),
    )
    def hist_kernel(widx_hbm, midx_hbm, zeros_hbm, wout_hbm, mout_hbm,
                    widx_v, midx_v, wcnt_v, mcnt_v):
        wid = lax.axis_index("s") * 2 + lax.axis_index("c")
        base = wid * _SLAB
        pltpu.sync_copy(widx_hbm.at[pl.ds(base, _SLAB)], widx_v)
        pltpu.sync_copy(midx_hbm.at[pl.ds(base, _SLAB)], midx_v)
        pltpu.sync_copy(zeros_hbm, wcnt_v)
        pltpu.sync_copy(zeros_hbm.at[pl.ds(0, _MFCC_EMB)], mcnt_v)
        for k in range(_SLAB // _L):
            iw = widx_v[pl.ds(k * _L, _L)]
            _hist_chunk(iw, wcnt_v)
            im = midx_v[pl.ds(k * _L, _L)] - _NUM_EMB
            _hist_chunk(im, mcnt_v)
        pltpu.sync_copy(wcnt_v, wout_hbm.at[wid])
        pltpu.sync_copy(mcnt_v, mout_hbm.at[wid])

    return hist_kernel(widx, midx, zeros)


def _finalize_body(wpart_ref, mpart_ref, wmin_ref, mmin_ref,
                   wperp_ref, wloss_ref, mperp_ref, mloss_ref):
    n = jnp.float32(_NUM_EMB)
    wp = jnp.sum(wpart_ref[...], axis=0) / n
    went = jnp.sum(wp * jnp.log(wp + 1e-10))
    wperp_ref[...] = jnp.exp(-went).reshape(1, 1)
    wloss = (1.0 + _COMMIT) * jnp.sum(wmin_ref[...]) / (n * _EMB_DIM)
    wloss_ref[...] = wloss.reshape(1, 1)

    mp = jnp.sum(mpart_ref[...], axis=0) / n
    ment = jnp.sum(mp * jnp.log(mp + 1e-10))
    mperp_ref[...] = jnp.exp(-ment).reshape(1, 1)
    mloss = (1.0 + _COMMIT) * jnp.sum(mmin_ref[...]) / (n * _MFCC_DIM)
    mloss_ref[...] = mloss.reshape(1, 1)


def _finalize(wpart, mpart, wmin, mmin):
    wmin2 = wmin.reshape(_NUM_EMB // 128, 128)
    mmin2 = mmin.reshape(_NUM_EMB // 128, 128)
    s = jax.ShapeDtypeStruct((1, 1), jnp.float32)
    return pl.pallas_call(
        _finalize_body,
        out_shape=[s, s, s, s],
        interpret=_INTERPRET,
    )(wpart, mpart, wmin2, mmin2)


def kernel(X, wave_codebook, mfcc_codebook):
    B, T = X.shape
    flat = X.reshape(-1, _EMB_DIM)
    n_idx = lax.iota(jnp.float32, _EMB_DIM)[:, None]
    k_idx = lax.iota(jnp.float32, _MFCC_DIM)[None, :] + 2.0
    c = jnp.cos((2.0 * jnp.pi / _EMB_DIM) * n_idx * k_idx)

    widx, wmin, midx, mmin = _vq_stage(flat,
                                       jnp.transpose(-2.0 * wave_codebook),
                                       jnp.transpose(-2.0 * mfcc_codebook),
                                       c)

    zeros = jnp.zeros((_NUM_EMB,), jnp.float32)
    wpart, mpart = _hist_sc(widx, midx, zeros)

    wperp, wloss, mperp, mloss = _finalize(wpart, mpart, wmin, mmin)

    N = T // _EMB_DIM
    w2 = widx.reshape(B, N)
    m2 = midx.reshape(B, N)
    encodings = jnp.stack([w2, m2], axis=2).reshape(B, 2 * N)
    return (encodings, wperp.reshape(()), wloss.reshape(()),
            mperp.reshape(()), mloss.reshape(()))
